# baseline (speedup 1.0000x reference)
import functools
import math

import jax
import jax.numpy as jnp
from jax import lax
from jax.experimental import pallas as pl
from jax.experimental.pallas import tpu as pltpu
from jax.experimental.pallas import tpu_sc as plsc

_V = 100000
_B = 1024
_SMOOTH = 0.1
_CONF = 1.0 - _SMOOTH
_EPS = _SMOOTH / (_V - 1)
_CONST = _B * ((_V - 1) * _EPS * math.log(_EPS) + _CONF * math.log(_CONF))

_ROWS = 32
_NW = 32
_RPW = _B // _NW
_L = 16


def _sum_body(x_ref, o_ref):
    i = pl.program_id(0)

    @pl.when(i == 0)
    def _():
        o_ref[0, 0] = jnp.float32(0.0)

    o_ref[0, 0] += jnp.sum(x_ref[...])


def _tc_sum(x):
    out = pl.pallas_call(
        _sum_body,
        grid=(_B // _ROWS,),
        in_specs=[pl.BlockSpec((_ROWS, _V), lambda i: (i, 0))],
        out_specs=pl.BlockSpec(memory_space=pltpu.SMEM),
        out_shape=jax.ShapeDtypeStruct((1, 1), jnp.float32),
    )(x)
    return out[0, 0]


def _sc_gather_body(xflat_hbm, tgt_hbm, out_hbm, tgt_v, idx_v, gath_v, acc_v, sem):
    wid = lax.axis_index("s") * 2 + lax.axis_index("c")
    base = wid * _RPW
    pltpu.sync_copy(tgt_hbm.at[pl.ds(base, _RPW)], tgt_v)
    for j in range(_RPW // _L):
        t = tgt_v[pl.ds(j * _L, _L)]
        row = base + j * _L + lax.iota(jnp.int32, _L)
        idx_v[pl.ds(j * _L, _L)] = row * _V + t
    pltpu.async_copy(xflat_hbm.at[idx_v], gath_v, sem).wait()
    acc = jnp.zeros((_L,), jnp.float32)
    for j in range(_RPW // _L):
        acc = acc + gath_v[pl.ds(j * _L, _L)]
    acc_v[...] = acc
    pltpu.sync_copy(acc_v, out_hbm.at[wid])


def _sc_gather(xflat, tgt):
    mesh = plsc.VectorSubcoreMesh(core_axis_name="c", subcore_axis_name="s")
    k = functools.partial(
        pl.kernel,
        mesh=mesh,
        out_type=jax.ShapeDtypeStruct((_NW, _L), jnp.float32),
        scratch_types=[
            pltpu.VMEM((_RPW,), jnp.int32),
            pltpu.VMEM((_RPW,), jnp.int32),
            pltpu.VMEM((_RPW,), jnp.float32),
            pltpu.VMEM((_L,), jnp.float32),
            pltpu.SemaphoreType.DMA,
        ],
    )(_sc_gather_body)
    return k(xflat, tgt)


def kernel(x, target):
    tgt = target.astype(jnp.int32)
    xflat = x.reshape(_B * _V)
    parts = _sc_gather(xflat, tgt)
    s = _tc_sum(x)
    g = jnp.sum(parts)
    return (jnp.float32(_CONST) - jnp.float32(_EPS) * s
            - jnp.float32(_CONF - _EPS) * g)


# speedup vs baseline: 2.2110x; 2.2110x over previous
import functools
import math

import jax
import jax.numpy as jnp
from jax import lax
from jax.experimental import pallas as pl
from jax.experimental.pallas import tpu as pltpu
from jax.experimental.pallas import tpu_sc as plsc

_V = 100000
_B = 1024
_SMOOTH = 0.1
_CONF = 1.0 - _SMOOTH
_EPS = _SMOOTH / (_V - 1)
_CONST = _B * ((_V - 1) * _EPS * math.log(_EPS) + _CONF * math.log(_CONF))

_NW = 32
_RPW = _B // _NW
_L = 16

_CROWS = 16
_NCHUNK = _B // _CROWS
_NBUF = 4


def _sum_body(x_hbm, o_ref, buf, sems):
    def start(c, b):
        pltpu.make_async_copy(
            x_hbm.at[pl.ds(c * _CROWS, _CROWS), :], buf.at[b], sems.at[b]
        ).start()

    def wait(c, b):
        pltpu.make_async_copy(
            x_hbm.at[pl.ds(c * _CROWS, _CROWS), :], buf.at[b], sems.at[b]
        ).wait()

    for b in range(_NBUF):
        start(b, b)

    def outer(g, acc):
        for b in range(_NBUF):
            c = g * _NBUF + b
            wait(c, b)
            acc = acc + jnp.sum(buf[b])

            @pl.when(c + _NBUF < _NCHUNK)
            def _():
                start(c + _NBUF, b)

        return acc

    o_ref[0, 0] = lax.fori_loop(
        0, _NCHUNK // _NBUF, outer, jnp.float32(0.0), unroll=False
    )


def _tc_sum(x):
    out = pl.pallas_call(
        _sum_body,
        in_specs=[pl.BlockSpec(memory_space=pl.ANY)],
        out_specs=pl.BlockSpec(memory_space=pltpu.SMEM),
        out_shape=jax.ShapeDtypeStruct((1, 1), jnp.float32),
        scratch_shapes=[
            pltpu.VMEM((_NBUF, _CROWS, _V), jnp.float32),
            pltpu.SemaphoreType.DMA((_NBUF,)),
        ],
    )(x)
    return out[0, 0]


def _sc_gather_body(x_hbm, tgt_hbm, out_hbm, tgt_v, gath_v, acc_v, sem):
    wid = lax.axis_index("s") * 2 + lax.axis_index("c")
    base = wid * _RPW
    pltpu.sync_copy(tgt_hbm.at[pl.ds(base, _RPW)], tgt_v)
    copies = []
    for grp in range(_RPW // _L):
        tv = tgt_v[pl.ds(grp * _L, _L)]
        for jj in range(_L):
            j = grp * _L + jj
            t = tv[jj]
            ct0 = pl.multiple_of(lax.bitwise_and(t, jnp.int32(~127)), 128)
            cp = pltpu.make_async_copy(
                x_hbm.at[pl.ds(base + (j // 8) * 8, 8), pl.ds(ct0, 128)],
                gath_v.at[j],
                sem,
            )
            cp.start()
            copies.append(cp)
    for cp in copies:
        cp.wait()
    acc = jnp.zeros((_L,), jnp.float32)
    lanes = lax.iota(jnp.int32, _L)
    for grp in range(_RPW // _L):
        tv = tgt_v[pl.ds(grp * _L, _L)]
        lanev = lax.bitwise_and(tv, jnp.int32(_L - 1))
        c0v = lax.bitwise_and(tv, jnp.int32(112))
        for jj in range(_L):
            j = grp * _L + jj
            vec = gath_v[j, j % 8, pl.ds(c0v[jj], _L)]
            acc = acc + jnp.where(lanes == lanev[jj], vec, jnp.float32(0.0))
    acc_v[...] = acc
    pltpu.sync_copy(acc_v, out_hbm.at[wid])


def _sc_gather(x, tgt):
    mesh = plsc.VectorSubcoreMesh(core_axis_name="c", subcore_axis_name="s")
    k = functools.partial(
        pl.kernel,
        mesh=mesh,
        out_type=jax.ShapeDtypeStruct((_NW, _L), jnp.float32),
        scratch_types=[
            pltpu.VMEM((_RPW,), jnp.int32),
            pltpu.VMEM((_RPW, 8, 128), jnp.float32),
            pltpu.VMEM((_L,), jnp.float32),
            pltpu.SemaphoreType.DMA,
        ],
    )(_sc_gather_body)
    return k(x, tgt)


def kernel(x, target):
    tgt = target.astype(jnp.int32)
    parts = _sc_gather(x, tgt)
    s = _tc_sum(x)
    g = jnp.sum(parts)
    return (jnp.float32(_CONST) - jnp.float32(_EPS) * s
            - jnp.float32(_CONF - _EPS) * g)
